# baseline (speedup 1.0000x reference)
import numpy as np
import jax
import jax.numpy as jnp
from jax.experimental import pallas as pl

NUM_FRAMES = 16
TUBELET = 2
PATCH = 16
EMBED_DIMS = 768


def _sinusoid_table(n_position, embed_dims, base=10000):
    vec = np.arange(embed_dims, dtype=np.float64)
    vec = (vec - vec % 2) / embed_dims
    vec = np.power(float(base), -vec).reshape(1, -1)
    table = np.arange(n_position, dtype=np.float64).reshape(-1, 1) * vec
    table[:, 0::2] = np.sin(table[:, 0::2])
    table[:, 1::2] = np.cos(table[:, 1::2])
    return table.astype(np.float32)


def _body(f_ref, pos_ref, tok_ref, fr_ref, pos_out_ref):
    f = f_ref[0]
    fc = f.reshape(TUBELET, PATCH, 7, 96)
    l = jax.lax.broadcasted_iota(jnp.int32, (TUBELET, PATCH, 7, 96), 3)
    c_of = l // 32
    wib_of = (l % 32) // PATCH
    p2_of = l % PATCH
    idx = 48 * wib_of + 3 * p2_of + c_of
    fd = jnp.take_along_axis(fc, idx, axis=-1)

    planes = [
        jax.lax.slice(fd, (0, 0, 0, 32 * c), (TUBELET, PATCH, 7, 32 * (c + 1)))
        .reshape(TUBELET, PATCH, 224)
        for c in range(3)
    ]
    fr_ref[0] = jnp.stack(planes, axis=1)

    m = fd.reshape(TUBELET * PATCH, 7 * 96)
    t2 = jnp.transpose(m, (1, 0))
    t3 = t2.reshape(7, 3, TUBELET, PATCH, 32)
    t3 = jnp.transpose(t3, (0, 2, 1, 3, 4))
    t4 = jnp.transpose(t3, (0, 1, 2, 4, 3))
    tok_ref[0, 0, 0] = t4.reshape(14, 3, 2 * PATCH, PATCH)

    pos_out_ref[0, 0] = pos_ref[0]


def kernel(frames, targets):
    B, T, H, W, C = frames.shape
    t = NUM_FRAMES // TUBELET
    h = H // PATCH
    w = W // PATCH
    total_tokens = t * h * w

    pos_table = jnp.asarray(
        _sinusoid_table(total_tokens, EMBED_DIMS).reshape(t, h * w, EMBED_DIMS)
    )
    frames_wc = frames.reshape(B, T, H, W * C)

    grid = (B, t, h)
    tok, fr_t, pos_out = pl.pallas_call(
        _body,
        grid=grid,
        in_specs=[
            pl.BlockSpec((1, TUBELET, PATCH, W * C), lambda b, i, j: (b, i, j, 0)),
            pl.BlockSpec((1, h * w, EMBED_DIMS), lambda b, i, j: (i, 0, 0)),
        ],
        out_specs=[
            pl.BlockSpec(
                (1, 1, 1, w, C, TUBELET * PATCH, PATCH),
                lambda b, i, j: (b, i, j, 0, 0, 0, 0),
            ),
            pl.BlockSpec(
                (1, TUBELET, C, PATCH, W), lambda b, i, j: (b, i, 0, j, 0)
            ),
            pl.BlockSpec((1, 1, h * w, EMBED_DIMS), lambda b, i, j: (b, i, 0, 0)),
        ],
        out_shape=[
            jax.ShapeDtypeStruct((B, t, h, w, C, TUBELET * PATCH, PATCH), frames.dtype),
            jax.ShapeDtypeStruct((B, T, C, H, W), frames.dtype),
            jax.ShapeDtypeStruct((B, t, h * w, EMBED_DIMS), jnp.float32),
        ],
    )(frames_wc, pos_table)

    split_crops_out = tok.reshape(B * total_tokens, C, TUBELET, PATCH, PATCH)
    pos_out = pos_out.reshape(B * total_tokens, EMBED_DIMS)
    num_valid_tokens = jnp.full((B,), total_tokens, dtype=jnp.int32)
    token_mask = jnp.ones((B, total_tokens), dtype=bool)
    return (split_crops_out, num_valid_tokens, pos_out, token_mask, fr_t)


# speedup vs baseline: 1.3984x; 1.3984x over previous
import numpy as np
import jax
import jax.numpy as jnp
from jax.experimental import pallas as pl

NUM_FRAMES = 16
TUBELET = 2
PATCH = 16
EMBED_DIMS = 768


def _sinusoid_table(n_position, embed_dims, base=10000):
    vec = np.arange(embed_dims, dtype=np.float64)
    vec = (vec - vec % 2) / embed_dims
    vec = np.power(float(base), -vec).reshape(1, -1)
    table = np.arange(n_position, dtype=np.float64).reshape(-1, 1) * vec
    table[:, 0::2] = np.sin(table[:, 0::2])
    table[:, 1::2] = np.cos(table[:, 1::2])
    return table.astype(np.float32)


def _body(f_ref, pos_ref, tok_ref, fr_ref, pos_out_ref):
    f = f_ref[0]
    fc = f.reshape(TUBELET, 224, 7, 96)
    l = jax.lax.broadcasted_iota(jnp.int32, (TUBELET, 224, 7, 96), 3)
    c_of = l // 32
    wib_of = (l % 32) // PATCH
    p2_of = l % PATCH
    idx = 48 * wib_of + 3 * p2_of + c_of
    fd = jnp.take_along_axis(fc, idx, axis=-1)

    planes = [
        jax.lax.slice(fd, (0, 0, 0, 32 * c), (TUBELET, 224, 7, 32 * (c + 1)))
        .reshape(TUBELET, 224, 224)
        for c in range(3)
    ]
    fr_ref[0] = jnp.stack(planes, axis=1)

    g = fd.reshape(TUBELET, 14, PATCH, 672)
    g = jnp.transpose(g, (1, 0, 2, 3))
    g = g.reshape(14, TUBELET * PATCH, 672)
    t = jnp.transpose(g, (0, 2, 1))
    t = t.reshape(14, 7, 3, TUBELET, PATCH, 32)
    t = jnp.transpose(t, (0, 1, 3, 2, 4, 5))
    t = jnp.transpose(t, (0, 1, 2, 3, 5, 4))
    tok_ref[0, 0] = t.reshape(14, 14, 3, TUBELET * PATCH, PATCH)

    pos_out_ref[0, 0] = pos_ref[0]


def kernel(frames, targets):
    B, T, H, W, C = frames.shape
    t = NUM_FRAMES // TUBELET
    h = H // PATCH
    w = W // PATCH
    total_tokens = t * h * w

    pos_table = jnp.asarray(
        _sinusoid_table(total_tokens, EMBED_DIMS).reshape(t, h * w, EMBED_DIMS)
    )
    frames_wc = frames.reshape(B, T, H, W * C)

    grid = (B, t)
    tok, fr_t, pos_out = pl.pallas_call(
        _body,
        grid=grid,
        in_specs=[
            pl.BlockSpec((1, TUBELET, H, W * C), lambda b, i: (b, i, 0, 0)),
            pl.BlockSpec((1, h * w, EMBED_DIMS), lambda b, i: (i, 0, 0)),
        ],
        out_specs=[
            pl.BlockSpec(
                (1, 1, h, w, C, TUBELET * PATCH, PATCH),
                lambda b, i: (b, i, 0, 0, 0, 0, 0),
            ),
            pl.BlockSpec((1, TUBELET, C, H, W), lambda b, i: (b, i, 0, 0, 0)),
            pl.BlockSpec((1, 1, h * w, EMBED_DIMS), lambda b, i: (b, i, 0, 0)),
        ],
        out_shape=[
            jax.ShapeDtypeStruct(
                (B, t, h, w, C, TUBELET * PATCH, PATCH), frames.dtype
            ),
            jax.ShapeDtypeStruct((B, T, C, H, W), frames.dtype),
            jax.ShapeDtypeStruct((B, t, h * w, EMBED_DIMS), jnp.float32),
        ],
    )(frames_wc, pos_table)

    split_crops_out = tok.reshape(B * total_tokens, C, TUBELET, PATCH, PATCH)
    pos_out = pos_out.reshape(B * total_tokens, EMBED_DIMS)
    num_valid_tokens = jnp.full((B,), total_tokens, dtype=jnp.int32)
    token_mask = jnp.ones((B, total_tokens), dtype=bool)
    return (split_crops_out, num_valid_tokens, pos_out, token_mask, fr_t)
